# Initial kernel scaffold; baseline (speedup 1.0000x reference)
import jax
import jax.numpy as jnp
from jax.experimental import pallas as pl


def kernel(x, keys, values):
    raise NotImplementedError("write your pallas kernel here")


# speedup vs baseline: 60.6102x; 60.6102x over previous
import functools

import jax
import jax.numpy as jnp
from jax import lax
from jax.experimental import pallas as pl
from jax.experimental.pallas import tpu as pltpu
from jax.experimental.pallas import tpu_sc as plsc


def _tc_argmax_body(x_ref, keys_ref, out_ref, *, n_keys):
    c = pl.program_id(0)
    f = x_ref[0, 0, :, :]
    k = keys_ref[0, :, :]
    f2 = jnp.sum(f * f, axis=-1)[:, None]
    fk = lax.dot_general(f, k, (((1,), (1,)), ((), ())),
                         preferred_element_type=jnp.float32)
    k2 = jnp.sum(k * k, axis=-1)[None, :]
    dist = -(f2 - 2.0 * fk + k2)
    m = jnp.max(dist, axis=1, keepdims=True)
    iota = lax.broadcasted_iota(jnp.int32, dist.shape, 1)
    cand = jnp.where(dist == m, iota, n_keys)
    idx = jnp.min(cand, axis=1)
    out_ref[0, 0, 0, :] = idx + c * n_keys


def _sc_gather(table_flat, gidx, n_rows_out, dv):
    info = plsc.get_sparse_core_info()
    nw = info.num_cores * info.num_subcores
    b_per_w = n_rows_out // nw
    ch = 128
    nch = b_per_w // ch
    mesh = plsc.VectorSubcoreMesh(core_axis_name="c", subcore_axis_name="s")

    @functools.partial(
        pl.kernel, mesh=mesh,
        out_type=jax.ShapeDtypeStruct((n_rows_out, dv), jnp.float32),
        scratch_types=[
            pltpu.VMEM((b_per_w,), jnp.int32),
            pltpu.VMEM((b_per_w, dv), jnp.float32),
            pltpu.SemaphoreType.DMA,
        ],
        compiler_params=pltpu.CompilerParams(use_tc_tiling_on_sc=False),
    )
    def k(table_hbm, idx_hbm, out_hbm, idx_v, rows_v, sem):
        wid = lax.axis_index("s") * info.num_cores + lax.axis_index("c")
        base = wid * b_per_w
        pltpu.sync_copy(idx_hbm.at[pl.ds(base, b_per_w)], idx_v)
        copies = []
        for j in range(nch):
            copies.append(pltpu.async_copy(
                table_hbm.at[idx_v.at[pl.ds(j * ch, ch)]],
                rows_v.at[pl.ds(j * ch, ch)],
                sem))
        for cp in copies:
            cp.wait()
        pltpu.sync_copy(rows_v, out_hbm.at[pl.ds(base, b_per_w)])

    return k(table_flat, gidx)


def kernel(x, keys, values):
    b, c, n, dk = x.shape
    k_keys = keys.shape[1]
    dv = values.shape[-1]

    gidx = pl.pallas_call(
        functools.partial(_tc_argmax_body, n_keys=k_keys),
        grid=(c, b),
        in_specs=[
            pl.BlockSpec((1, 1, n, dk), lambda ci, bi: (bi, ci, 0, 0)),
            pl.BlockSpec((1, k_keys, dk), lambda ci, bi: (ci, 0, 0)),
        ],
        out_specs=pl.BlockSpec((1, 1, 1, n), lambda ci, bi: (bi, ci, 0, 0)),
        out_shape=jax.ShapeDtypeStruct((b, c, 1, n), jnp.int32),
    )(x, keys)

    out_flat = _sc_gather(
        values.reshape(c * k_keys, dv),
        gidx.reshape(b * c * n),
        b * c * n,
        dv,
    )
    return (out_flat.reshape(b, c, 1, n, dv),)


# speedup vs baseline: 64.4361x; 1.0631x over previous
import functools

import jax
import jax.numpy as jnp
from jax import lax
from jax.experimental import pallas as pl
from jax.experimental.pallas import tpu as pltpu
from jax.experimental.pallas import tpu_sc as plsc


def _tc_argmax_body(x_ref, keys_ref, out_ref, k2_ref, *, n_keys):
    c = pl.program_id(0)
    f = x_ref[0, 0, :, :]
    k = keys_ref[0, :, :]

    @pl.when(pl.program_id(1) == 0)
    def _():
        k2_ref[0, :] = jnp.sum(k * k, axis=-1)

    f2 = jnp.sum(f * f, axis=-1)[:, None]
    fk2 = lax.dot_general(f + f, k, (((1,), (1,)), ((), ())),
                          preferred_element_type=jnp.float32)
    t = (f2 - fk2) + k2_ref[0, :][None, :]
    m = jnp.min(t, axis=1, keepdims=True)
    iota = lax.broadcasted_iota(jnp.int32, t.shape, 1)
    cand = jnp.where(t == m, iota, n_keys)
    idx = jnp.min(cand, axis=1)
    out_ref[0, 0, 0, :] = idx + c * n_keys


def _sc_gather(table_flat, gidx, n_rows_out, dv):
    info = plsc.get_sparse_core_info()
    nw = info.num_cores * info.num_subcores
    b_per_w = n_rows_out // nw
    ch = 128
    nch = b_per_w // ch
    mesh = plsc.VectorSubcoreMesh(core_axis_name="c", subcore_axis_name="s")

    @functools.partial(
        pl.kernel, mesh=mesh,
        out_type=jax.ShapeDtypeStruct((n_rows_out, dv), jnp.float32),
        scratch_types=[
            pltpu.VMEM((b_per_w,), jnp.int32),
            pltpu.VMEM((b_per_w, dv), jnp.float32),
            pltpu.SemaphoreType.DMA,
        ],
        compiler_params=pltpu.CompilerParams(use_tc_tiling_on_sc=False),
    )
    def k(table_hbm, idx_hbm, out_hbm, idx_v, rows_v, sem):
        wid = lax.axis_index("s") * info.num_cores + lax.axis_index("c")
        base = wid * b_per_w
        pltpu.sync_copy(idx_hbm.at[pl.ds(base, b_per_w)], idx_v)
        copies = []
        for j in range(nch):
            copies.append(pltpu.async_copy(
                table_hbm.at[idx_v.at[pl.ds(j * ch, ch)]],
                rows_v.at[pl.ds(j * ch, ch)],
                sem))
        for cp in copies:
            cp.wait()
        pltpu.sync_copy(rows_v, out_hbm.at[pl.ds(base, b_per_w)])

    return k(table_flat, gidx)


def kernel(x, keys, values):
    b, c, n, dk = x.shape
    k_keys = keys.shape[1]
    dv = values.shape[-1]

    gidx = pl.pallas_call(
        functools.partial(_tc_argmax_body, n_keys=k_keys),
        grid=(c, b),
        in_specs=[
            pl.BlockSpec((1, 1, n, dk), lambda ci, bi: (bi, ci, 0, 0)),
            pl.BlockSpec((1, k_keys, dk), lambda ci, bi: (ci, 0, 0)),
        ],
        out_specs=pl.BlockSpec((1, 1, 1, n), lambda ci, bi: (bi, ci, 0, 0)),
        out_shape=jax.ShapeDtypeStruct((b, c, 1, n), jnp.int32),
        scratch_shapes=[pltpu.VMEM((1, k_keys), jnp.float32)],
    )(x, keys)

    out_flat = _sc_gather(
        values.reshape(c * k_keys, dv),
        gidx.reshape(b * c * n),
        b * c * n,
        dv,
    )
    return (out_flat.reshape(b, c, 1, n, dv),)


# speedup vs baseline: 68.4678x; 1.0626x over previous
import functools

import jax
import jax.numpy as jnp
from jax import lax
from jax.experimental import pallas as pl
from jax.experimental.pallas import tpu as pltpu
from jax.experimental.pallas import tpu_sc as plsc


def _tc_argmax_body(x_ref, keys_ref, out_ref, k2_ref, *, n_keys, bb):
    c = pl.program_id(0)
    n = x_ref.shape[2]
    dk = x_ref.shape[3]
    f = x_ref[:, 0, :, :].reshape(bb * n, dk)
    k = keys_ref[0, :, :]

    @pl.when(pl.program_id(1) == 0)
    def _():
        k2_ref[0, :] = jnp.sum(k * k, axis=-1)

    f2 = jnp.sum(f * f, axis=-1)[:, None]
    fk2 = lax.dot_general(f + f, k, (((1,), (1,)), ((), ())),
                          preferred_element_type=jnp.float32)
    t = (f2 - fk2) + k2_ref[0, :][None, :]
    m = jnp.min(t, axis=1, keepdims=True)
    iota = lax.broadcasted_iota(jnp.int32, t.shape, 1)
    cand = jnp.where(t == m, iota, n_keys)
    idx = jnp.min(cand, axis=1) + c * n_keys
    out_ref[:, 0, 0, :] = idx.reshape(bb, n)


def _sc_gather(table_flat, gidx, n_rows_out, dv):
    info = plsc.get_sparse_core_info()
    nw = info.num_cores * info.num_subcores
    b_per_w = n_rows_out // nw
    ch = 128
    nch = b_per_w // ch
    mesh = plsc.VectorSubcoreMesh(core_axis_name="c", subcore_axis_name="s")

    @functools.partial(
        pl.kernel, mesh=mesh,
        out_type=jax.ShapeDtypeStruct((n_rows_out, dv), jnp.float32),
        scratch_types=[
            pltpu.VMEM((b_per_w,), jnp.int32),
            pltpu.VMEM((b_per_w, dv), jnp.float32),
            pltpu.SemaphoreType.DMA,
        ],
        compiler_params=pltpu.CompilerParams(use_tc_tiling_on_sc=False),
    )
    def k(table_hbm, idx_hbm, out_hbm, idx_v, rows_v, sem):
        wid = lax.axis_index("s") * info.num_cores + lax.axis_index("c")
        base = wid * b_per_w
        pltpu.sync_copy(idx_hbm.at[pl.ds(base, b_per_w)], idx_v)
        copies = []
        for j in range(nch):
            copies.append(pltpu.async_copy(
                table_hbm.at[idx_v.at[pl.ds(j * ch, ch)]],
                rows_v.at[pl.ds(j * ch, ch)],
                sem))
        for cp in copies:
            cp.wait()
        pltpu.sync_copy(rows_v, out_hbm.at[pl.ds(base, b_per_w)])

    return k(table_flat, gidx)


def kernel(x, keys, values):
    b, c, n, dk = x.shape
    k_keys = keys.shape[1]
    dv = values.shape[-1]

    bb = 2
    gidx = pl.pallas_call(
        functools.partial(_tc_argmax_body, n_keys=k_keys, bb=bb),
        grid=(c, b // bb),
        in_specs=[
            pl.BlockSpec((bb, 1, n, dk), lambda ci, bi: (bi, ci, 0, 0)),
            pl.BlockSpec((1, k_keys, dk), lambda ci, bi: (ci, 0, 0)),
        ],
        out_specs=pl.BlockSpec((bb, 1, 1, n), lambda ci, bi: (bi, ci, 0, 0)),
        out_shape=jax.ShapeDtypeStruct((b, c, 1, n), jnp.int32),
        scratch_shapes=[pltpu.VMEM((1, k_keys), jnp.float32)],
    )(x, keys)

    out_flat = _sc_gather(
        values.reshape(c * k_keys, dv),
        gidx.reshape(b * c * n),
        b * c * n,
        dv,
    )
    return (out_flat.reshape(b, c, 1, n, dv),)


# speedup vs baseline: 72.3079x; 1.0561x over previous
import functools

import jax
import jax.numpy as jnp
from jax import lax
from jax.experimental import pallas as pl
from jax.experimental.pallas import tpu as pltpu
from jax.experimental.pallas import tpu_sc as plsc


def _tc_argmax_body(x_ref, keys_ref, out_ref, k2_ref, *, n_keys, bb):
    c = pl.program_id(0)
    n = x_ref.shape[2]
    dk = x_ref.shape[3]
    f = x_ref[:, 0, :, :].reshape(bb * n, dk)
    k = keys_ref[0, :, :]

    @pl.when(pl.program_id(1) == 0)
    def _():
        k2_ref[0, :] = jnp.sum(k * k, axis=-1)

    f2 = jnp.sum(f * f, axis=-1)[:, None]
    fk2 = lax.dot_general(f + f, k, (((1,), (1,)), ((), ())),
                          preferred_element_type=jnp.float32)
    t = (f2 - fk2) + k2_ref[0, :][None, :]
    m = jnp.min(t, axis=1, keepdims=True)
    iota = lax.broadcasted_iota(jnp.int32, t.shape, 1)
    cand = jnp.where(t == m, iota, n_keys)
    idx = jnp.min(cand, axis=1) + c * n_keys
    out_ref[:, 0, 0, :] = idx.reshape(bb, n)


def _sc_gather(table_flat, gidx, n_rows_out, dv):
    info = plsc.get_sparse_core_info()
    nw = info.num_cores * info.num_subcores
    b_per_w = n_rows_out // nw
    ch = 128
    nch = b_per_w // ch
    mesh = plsc.VectorSubcoreMesh(core_axis_name="c", subcore_axis_name="s")

    @functools.partial(
        pl.kernel, mesh=mesh,
        out_type=jax.ShapeDtypeStruct((n_rows_out, dv), jnp.float32),
        scratch_types=[
            pltpu.VMEM((b_per_w,), jnp.int32),
            pltpu.VMEM((b_per_w, dv), jnp.float32),
            pltpu.SemaphoreType.DMA,
        ],
        compiler_params=pltpu.CompilerParams(use_tc_tiling_on_sc=False),
    )
    def k(table_hbm, idx_hbm, out_hbm, idx_v, rows_v, sem):
        wid = lax.axis_index("s") * info.num_cores + lax.axis_index("c")
        base = wid * b_per_w
        pltpu.sync_copy(idx_hbm.at[pl.ds(base, b_per_w)], idx_v)
        copies = []
        for j in range(nch):
            copies.append(pltpu.async_copy(
                table_hbm.at[idx_v.at[pl.ds(j * ch, ch)]],
                rows_v.at[pl.ds(j * ch, ch)],
                sem))
        for cp in copies:
            cp.wait()
        pltpu.sync_copy(rows_v, out_hbm.at[pl.ds(base, b_per_w)])

    return k(table_flat, gidx)


def kernel(x, keys, values):
    b, c, n, dk = x.shape
    k_keys = keys.shape[1]
    dv = values.shape[-1]

    bb = 4
    gidx = pl.pallas_call(
        functools.partial(_tc_argmax_body, n_keys=k_keys, bb=bb),
        grid=(c, b // bb),
        in_specs=[
            pl.BlockSpec((bb, 1, n, dk), lambda ci, bi: (bi, ci, 0, 0)),
            pl.BlockSpec((1, k_keys, dk), lambda ci, bi: (ci, 0, 0)),
        ],
        out_specs=pl.BlockSpec((bb, 1, 1, n), lambda ci, bi: (bi, ci, 0, 0)),
        out_shape=jax.ShapeDtypeStruct((b, c, 1, n), jnp.int32),
        scratch_shapes=[pltpu.VMEM((1, k_keys), jnp.float32)],
    )(x, keys)

    out_flat = _sc_gather(
        values.reshape(c * k_keys, dv),
        gidx.reshape(b * c * n),
        b * c * n,
        dv,
    )
    return (out_flat.reshape(b, c, 1, n, dv),)


# speedup vs baseline: 76.2209x; 1.0541x over previous
import functools

import jax
import jax.numpy as jnp
from jax import lax
from jax.experimental import pallas as pl
from jax.experimental.pallas import tpu as pltpu
from jax.experimental.pallas import tpu_sc as plsc


def _tc_argmax_body(x_ref, keys_ref, out_ref, k2_ref, *, n_keys, bb):
    c = pl.program_id(0)
    n = x_ref.shape[2]
    dk = x_ref.shape[3]
    f = x_ref[:, 0, :, :].reshape(bb * n, dk)
    k = keys_ref[0, :, :]

    @pl.when(pl.program_id(1) == 0)
    def _():
        k2_ref[0, :] = jnp.sum(k * k, axis=-1)

    f2 = jnp.sum(f * f, axis=-1)[:, None]
    fk2 = lax.dot_general(f + f, k, (((1,), (1,)), ((), ())),
                          preferred_element_type=jnp.float32)
    t = (f2 - fk2) + k2_ref[0, :][None, :]
    m = jnp.min(t, axis=1, keepdims=True)
    iota = lax.broadcasted_iota(jnp.int32, t.shape, 1)
    cand = jnp.where(t == m, iota, n_keys)
    idx = jnp.min(cand, axis=1) + c * n_keys
    out_ref[:, 0, 0, :] = idx.reshape(bb, n)


def _sc_gather(table_flat, gidx, n_rows_out, dv):
    info = plsc.get_sparse_core_info()
    nw = info.num_cores * info.num_subcores
    b_per_w = n_rows_out // nw
    ch = 128
    nch = b_per_w // ch
    mesh = plsc.VectorSubcoreMesh(core_axis_name="c", subcore_axis_name="s")

    @functools.partial(
        pl.kernel, mesh=mesh,
        out_type=jax.ShapeDtypeStruct((n_rows_out, dv), jnp.float32),
        scratch_types=[
            pltpu.VMEM((b_per_w,), jnp.int32),
            pltpu.VMEM((b_per_w, dv), jnp.float32),
            pltpu.SemaphoreType.DMA,
        ],
        compiler_params=pltpu.CompilerParams(use_tc_tiling_on_sc=False),
    )
    def k(table_hbm, idx_hbm, out_hbm, idx_v, rows_v, sem):
        wid = lax.axis_index("s") * info.num_cores + lax.axis_index("c")
        base = wid * b_per_w
        pltpu.sync_copy(idx_hbm.at[pl.ds(base, b_per_w)], idx_v)
        copies = []
        for j in range(nch):
            copies.append(pltpu.async_copy(
                table_hbm.at[idx_v.at[pl.ds(j * ch, ch)]],
                rows_v.at[pl.ds(j * ch, ch)],
                sem))
        for cp in copies:
            cp.wait()
        pltpu.sync_copy(rows_v, out_hbm.at[pl.ds(base, b_per_w)])

    return k(table_flat, gidx)


def kernel(x, keys, values):
    b, c, n, dk = x.shape
    k_keys = keys.shape[1]
    dv = values.shape[-1]

    bb = 8
    gidx = pl.pallas_call(
        functools.partial(_tc_argmax_body, n_keys=k_keys, bb=bb),
        grid=(c, b // bb),
        in_specs=[
            pl.BlockSpec((bb, 1, n, dk), lambda ci, bi: (bi, ci, 0, 0)),
            pl.BlockSpec((1, k_keys, dk), lambda ci, bi: (ci, 0, 0)),
        ],
        out_specs=pl.BlockSpec((bb, 1, 1, n), lambda ci, bi: (bi, ci, 0, 0)),
        out_shape=jax.ShapeDtypeStruct((b, c, 1, n), jnp.int32),
        scratch_shapes=[pltpu.VMEM((1, k_keys), jnp.float32)],
    )(x, keys)

    out_flat = _sc_gather(
        values.reshape(c * k_keys, dv),
        gidx.reshape(b * c * n),
        b * c * n,
        dv,
    )
    return (out_flat.reshape(b, c, 1, n, dv),)


# speedup vs baseline: 82.2450x; 1.0790x over previous
import functools

import jax
import jax.numpy as jnp
from jax import lax
from jax.experimental import pallas as pl
from jax.experimental.pallas import tpu as pltpu
from jax.experimental.pallas import tpu_sc as plsc


def _tc_argmax_body(x_ref, keys_ref, out_ref, k2_ref, *, n_keys, bb):
    c = pl.program_id(0)
    n = x_ref.shape[2]
    dk = x_ref.shape[3]
    f = x_ref[:, 0, :, :].reshape(bb * n, dk)
    k = keys_ref[0, :, :]

    @pl.when(pl.program_id(1) == 0)
    def _():
        k2_ref[0, :] = jnp.sum(k * k, axis=-1)

    f2 = jnp.sum(f * f, axis=-1)[:, None]
    fk2 = lax.dot_general(f + f, k, (((1,), (1,)), ((), ())),
                          preferred_element_type=jnp.float32)
    t = (f2 - fk2) + k2_ref[0, :][None, :]
    idx = jnp.argmin(t, axis=1).astype(jnp.int32) + c * n_keys
    out_ref[:, 0, 0, :] = idx.reshape(bb, n)


def _sc_gather(table_flat, gidx, n_rows_out, dv):
    info = plsc.get_sparse_core_info()
    nw = info.num_cores * info.num_subcores
    b_per_w = n_rows_out // nw
    ch = 128
    nch = b_per_w // ch
    mesh = plsc.VectorSubcoreMesh(core_axis_name="c", subcore_axis_name="s")

    @functools.partial(
        pl.kernel, mesh=mesh,
        out_type=jax.ShapeDtypeStruct((n_rows_out, dv), jnp.float32),
        scratch_types=[
            pltpu.VMEM((b_per_w,), jnp.int32),
            pltpu.VMEM((b_per_w, dv), jnp.float32),
            pltpu.SemaphoreType.DMA,
        ],
        compiler_params=pltpu.CompilerParams(use_tc_tiling_on_sc=False),
    )
    def k(table_hbm, idx_hbm, out_hbm, idx_v, rows_v, sem):
        wid = lax.axis_index("s") * info.num_cores + lax.axis_index("c")
        base = wid * b_per_w
        pltpu.sync_copy(idx_hbm.at[pl.ds(base, b_per_w)], idx_v)
        copies = []
        for j in range(nch):
            copies.append(pltpu.async_copy(
                table_hbm.at[idx_v.at[pl.ds(j * ch, ch)]],
                rows_v.at[pl.ds(j * ch, ch)],
                sem))
        for cp in copies:
            cp.wait()
        pltpu.sync_copy(rows_v, out_hbm.at[pl.ds(base, b_per_w)])

    return k(table_flat, gidx)


def kernel(x, keys, values):
    b, c, n, dk = x.shape
    k_keys = keys.shape[1]
    dv = values.shape[-1]

    bb = 8
    gidx = pl.pallas_call(
        functools.partial(_tc_argmax_body, n_keys=k_keys, bb=bb),
        grid=(c, b // bb),
        in_specs=[
            pl.BlockSpec((bb, 1, n, dk), lambda ci, bi: (bi, ci, 0, 0)),
            pl.BlockSpec((1, k_keys, dk), lambda ci, bi: (ci, 0, 0)),
        ],
        out_specs=pl.BlockSpec((bb, 1, 1, n), lambda ci, bi: (bi, ci, 0, 0)),
        out_shape=jax.ShapeDtypeStruct((b, c, 1, n), jnp.int32),
        scratch_shapes=[pltpu.VMEM((1, k_keys), jnp.float32)],
    )(x, keys)

    out_flat = _sc_gather(
        values.reshape(c * k_keys, dv),
        gidx.reshape(b * c * n),
        b * c * n,
        dv,
    )
    return (out_flat.reshape(b, c, 1, n, dv),)


# speedup vs baseline: 82.6557x; 1.0050x over previous
import functools

import jax
import jax.numpy as jnp
from jax import lax
from jax.experimental import pallas as pl
from jax.experimental.pallas import tpu as pltpu
from jax.experimental.pallas import tpu_sc as plsc


def _tc_argmax_body(x_ref, keys_ref, out_ref, k2_ref, *, n_keys, bb):
    c = pl.program_id(0)
    n = x_ref.shape[2]
    dk = x_ref.shape[3]
    f = x_ref[:, 0, :, :].reshape(bb * n, dk)
    k = keys_ref[0, :, :]

    @pl.when(pl.program_id(1) == 0)
    def _():
        k2_ref[0, :] = jnp.sum(k * k, axis=-1)

    f2 = jnp.sum(f * f, axis=-1)[:, None]
    fk2 = lax.dot_general(f + f, k, (((1,), (1,)), ((), ())),
                          preferred_element_type=jnp.float32)
    t = (f2 - fk2) + k2_ref[0, :][None, :]
    idx = jnp.argmin(t, axis=1).astype(jnp.int32) + c * n_keys
    out_ref[:, 0, 0, :] = idx.reshape(bb, n)


def _sc_gather(table_flat, gidx, n_rows_out, dv):
    info = plsc.get_sparse_core_info()
    nw = info.num_cores * info.num_subcores
    b_per_w = n_rows_out // nw
    ch = 128
    nch = b_per_w // ch
    mesh = plsc.VectorSubcoreMesh(core_axis_name="c", subcore_axis_name="s")

    @functools.partial(
        pl.kernel, mesh=mesh,
        out_type=jax.ShapeDtypeStruct((n_rows_out, dv), jnp.float32),
        scratch_types=[
            pltpu.VMEM((b_per_w,), jnp.int32),
            pltpu.VMEM((b_per_w, dv), jnp.float32),
            pltpu.SemaphoreType.DMA,
        ],
        compiler_params=pltpu.CompilerParams(use_tc_tiling_on_sc=False),
    )
    def k(table_hbm, idx_hbm, out_hbm, idx_v, rows_v, sem):
        wid = lax.axis_index("s") * info.num_cores + lax.axis_index("c")
        base = wid * b_per_w
        pltpu.sync_copy(idx_hbm.at[pl.ds(base, b_per_w)], idx_v)
        copies = []
        for j in range(nch):
            copies.append(pltpu.async_copy(
                table_hbm.at[idx_v.at[pl.ds(j * ch, ch)]],
                rows_v.at[pl.ds(j * ch, ch)],
                sem))
        for cp in copies:
            cp.wait()
        pltpu.sync_copy(rows_v, out_hbm.at[pl.ds(base, b_per_w)])

    return k(table_flat, gidx)


def kernel(x, keys, values):
    b, c, n, dk = x.shape
    k_keys = keys.shape[1]
    dv = values.shape[-1]

    bb = 16
    gidx = pl.pallas_call(
        functools.partial(_tc_argmax_body, n_keys=k_keys, bb=bb),
        grid=(c, b // bb),
        in_specs=[
            pl.BlockSpec((bb, 1, n, dk), lambda ci, bi: (bi, ci, 0, 0)),
            pl.BlockSpec((1, k_keys, dk), lambda ci, bi: (ci, 0, 0)),
        ],
        out_specs=pl.BlockSpec((bb, 1, 1, n), lambda ci, bi: (bi, ci, 0, 0)),
        out_shape=jax.ShapeDtypeStruct((b, c, 1, n), jnp.int32),
        scratch_shapes=[pltpu.VMEM((1, k_keys), jnp.float32)],
    )(x, keys)

    out_flat = _sc_gather(
        values.reshape(c * k_keys, dv),
        gidx.reshape(b * c * n),
        b * c * n,
        dv,
    )
    return (out_flat.reshape(b, c, 1, n, dv),)
